# baseline (speedup 1.0000x reference)
import jax
import jax.numpy as jnp
from jax.experimental import pallas as pl
from jax.experimental.pallas import tpu as pltpu

BINS = 100
LO, HI = -5.0, 5.0
WIDTH = (HI - LO) / BINS
ENT_THRESH = 2.5
EPS = 1e-9
SQUARINGS = 5


def _main_body(e_ref, st_ref, adj_ref, w1_ref, w1t_ref, b1_ref, w2_ref,
               b2_ref, scale_ref, out_ref, ent_ref):
    x = e_ref[...]
    r, d = x.shape
    n_chunks = d // 128

    nib = jnp.int32(0x0F0F0F0F)
    m16 = jnp.int32(0x00FF00FF)

    def grp_body(gi, carry):
        g8 = pl.multiple_of(gi * 8, 8)
        longs = [jnp.zeros((8, 128), jnp.int32) for _ in range(26)]
        for r0 in range(0, n_chunks, 15):
            saccs = [jnp.zeros((8, 128), jnp.int32) for _ in range(13)]
            for c in range(r0, min(r0 + 15, n_chunks)):
                xc = e_ref[pl.ds(g8, 8), c * 128:(c + 1) * 128]
                t = (xc - LO) * jnp.float32(1.0 / WIDTH)
                ii = t.astype(jnp.int32)
                valid = jnp.abs(xc) <= HI
                idxs = jnp.where(valid, jnp.minimum(ii, BINS - 1), 1000)
                qq = jax.lax.shift_right_logical(idxs, 3)
                sv = jax.lax.shift_left(
                    jnp.int32(1),
                    jax.lax.shift_left(jnp.bitwise_and(idxs, 7), 2))
                for k in range(13):
                    saccs[k] = saccs[k] + jnp.where(qq == k, sv, 0)
            for k in range(13):
                longs[2 * k] = longs[2 * k] + jnp.bitwise_and(saccs[k], nib)
                longs[2 * k + 1] = longs[2 * k + 1] + jnp.bitwise_and(
                    jax.lax.shift_right_logical(saccs[k], 4), nib)
        tot = longs[0]
        for k in range(1, 26):
            tot = tot + longs[k]
        sp = (jnp.sum(jnp.bitwise_and(tot, m16), axis=1, keepdims=True) +
              jnp.sum(jnp.bitwise_and(jax.lax.shift_right_logical(tot, 8), m16),
                      axis=1, keepdims=True))
        s_tot = (jnp.bitwise_and(sp, 0xFFFF) +
                 jax.lax.shift_right_logical(sp, 16)).astype(jnp.float32)
        r_s = 1.0 / (s_tot + EPS)
        acc_h = jnp.zeros((8, 1), jnp.float32)
        for k in range(26):
            lo_s = jnp.sum(jnp.bitwise_and(longs[k], m16),
                           axis=1, keepdims=True)
            hi_s = jnp.sum(jnp.bitwise_and(
                jax.lax.shift_right_logical(longs[k], 8), m16),
                axis=1, keepdims=True)
            for half in (jnp.bitwise_and(lo_s, 0xFFFF),
                         jnp.bitwise_and(hi_s, 0xFFFF),
                         jax.lax.shift_right_logical(lo_s, 16),
                         jax.lax.shift_right_logical(hi_s, 16)):
                p = half.astype(jnp.float32) * r_s
                acc_h = acc_h + p * jnp.log(p + EPS)
        ent_ref[pl.ds(g8, 8), :] = -acc_h
        return carry

    jax.lax.fori_loop(0, r // 8, grp_body, 0)
    passed = ent_ref[...] >= ENT_THRESH

    comp = jnp.dot(x, st_ref[...], preferred_element_type=jnp.float32)
    filt = jnp.where(passed, comp, 0.0)
    state = jnp.dot(filt, adj_ref[...], preferred_element_type=jnp.float32)

    pre1 = jnp.dot(state, w1_ref[...], preferred_element_type=jnp.float32) + b1_ref[...]
    relu_mask = (pre1 > 0).astype(jnp.float32)
    c_row = (jnp.sum(w2_ref[...], axis=1) * scale_ref[0]).reshape(1, -1)
    g = jnp.dot(relu_mask * c_row, w1t_ref[...], preferred_element_type=jnp.float32)
    s_adv = state - 0.1 * jnp.sign(g)
    hid = jnp.maximum(
        jnp.dot(s_adv, w1_ref[...], preferred_element_type=jnp.float32) + b1_ref[...],
        0.0)
    logits = jnp.dot(hid, w2_ref[...], preferred_element_type=jnp.float32) + b2_ref[...]
    mx = jnp.max(logits, axis=1, keepdims=True)
    ex = jnp.exp(logits - mx)
    out_ref[...] = ex / jnp.sum(ex, axis=1, keepdims=True)


def _expm_body(adj_ref, out_ref):
    a = adj_ref[...]
    n = a.shape[0]
    t = (a * a) * (2.0 ** -SQUARINGS)
    eye = (jax.lax.broadcasted_iota(jnp.int32, (n, n), 0) ==
           jax.lax.broadcasted_iota(jnp.int32, (n, n), 1)).astype(jnp.float32)
    p = eye + t * (1.0 / 6.0)
    for k in (5.0, 4.0, 3.0, 2.0):
        p = eye + jnp.dot(t, p, preferred_element_type=jnp.float32,
                          precision=jax.lax.Precision.HIGHEST) * (1.0 / k)
    e = jnp.dot(t, p, preferred_element_type=jnp.float32,
                precision=jax.lax.Precision.HIGHEST)
    for _ in range(SQUARINGS):
        e = 2.0 * e + jnp.dot(e, e, preferred_element_type=jnp.float32,
                              precision=jax.lax.Precision.HIGHEST)
    tr_col = jnp.sum(e * eye, axis=1, keepdims=True)
    out_ref[...] = jnp.sum(tr_col, axis=0, keepdims=True)


def kernel(event_stream, sensing_matrix, adjacency, w1, b1, w2, b2):
    b_dim, d_dim = event_stream.shape
    n_dim = sensing_matrix.shape[0]
    h_dim = w1.shape[1]
    a_dim = w2.shape[1]
    r = 256 if b_dim % 256 == 0 else b_dim
    grid = b_dim // r

    st = sensing_matrix.T
    w1t = w1.T
    b1r = b1.reshape(1, h_dim)
    b2r = b2.reshape(1, a_dim)
    scale = jnp.full((1,), 1.0 / (b_dim * a_dim), jnp.float32)

    probs = pl.pallas_call(
        _main_body,
        grid=(grid,),
        in_specs=[
            pl.BlockSpec((r, d_dim), lambda i: (i, 0)),
            pl.BlockSpec((d_dim, n_dim), lambda i: (0, 0)),
            pl.BlockSpec((n_dim, n_dim), lambda i: (0, 0)),
            pl.BlockSpec((n_dim, h_dim), lambda i: (0, 0)),
            pl.BlockSpec((h_dim, n_dim), lambda i: (0, 0)),
            pl.BlockSpec((1, h_dim), lambda i: (0, 0)),
            pl.BlockSpec((h_dim, a_dim), lambda i: (0, 0)),
            pl.BlockSpec((1, a_dim), lambda i: (0, 0)),
            pl.BlockSpec(memory_space=pltpu.SMEM),
        ],
        out_specs=pl.BlockSpec((r, a_dim), lambda i: (i, 0)),
        out_shape=jax.ShapeDtypeStruct((b_dim, a_dim), jnp.float32),
        scratch_shapes=[pltpu.VMEM((r, 1), jnp.float32)],
        compiler_params=pltpu.CompilerParams(
            dimension_semantics=("arbitrary",),
            vmem_limit_bytes=56 * 1024 * 1024,
        ),
        name="acie_fused_main",
    )(event_stream, st, adjacency, w1, w1t, b1r, w2, b2r, scale)

    tr = pl.pallas_call(
        _expm_body,
        out_shape=jax.ShapeDtypeStruct((1, 1), jnp.float32),
        name="acie_expm_trace",
    )(adjacency)

    return (probs, tr[0, 0])


# speedup vs baseline: 1.0708x; 1.0708x over previous
import jax
import jax.numpy as jnp
from jax.experimental import pallas as pl
from jax.experimental.pallas import tpu as pltpu

BINS = 100
LO, HI = -5.0, 5.0
WIDTH = (HI - LO) / BINS
ENT_THRESH = 2.5
EPS = 1e-9
SQUARINGS = 5


def _main_body(e_ref, st_ref, adj_ref, w1_ref, w1t_ref, b1_ref, w2_ref,
               b2_ref, scale_ref, out_ref, ent_ref):
    x = e_ref[...]
    r, d = x.shape
    n_chunks = d // 128

    nib = jnp.int32(0x0F0F0F0F)
    m16 = jnp.int32(0x00FF00FF)

    def grp_body(gi, carry):
        g8 = pl.multiple_of(gi * 8, 8)
        longs = [jnp.zeros((8, 128), jnp.int32) for _ in range(26)]
        vcnt = jnp.zeros((8, 128), jnp.int32)
        for r0 in range(0, n_chunks, 15):
            saccs = [jnp.zeros((8, 128), jnp.int32) for _ in range(13)]
            for c in range(r0, min(r0 + 15, n_chunks)):
                xc = e_ref[pl.ds(g8, 8), c * 128:(c + 1) * 128]
                t = (xc - LO) * jnp.float32(1.0 / WIDTH)
                ii = t.astype(jnp.int32)
                valid = jnp.abs(xc) <= HI
                idxs = jnp.where(valid, jnp.minimum(ii, BINS - 1), 1000)
                qq = jax.lax.shift_right_logical(idxs, 3)
                sv = jax.lax.shift_left(
                    jnp.int32(1),
                    jax.lax.shift_left(jnp.bitwise_and(idxs, 7), 2))
                vcnt = vcnt + jnp.where(valid, 1, 0)
                for k in range(13):
                    saccs[k] = saccs[k] + jnp.where(qq == k, sv, 0)
            for k in range(13):
                longs[2 * k] = longs[2 * k] + jnp.bitwise_and(saccs[k], nib)
                longs[2 * k + 1] = longs[2 * k + 1] + jnp.bitwise_and(
                    jax.lax.shift_right_logical(saccs[k], 4), nib)
        s_tot = jnp.sum(vcnt, axis=1, keepdims=True).astype(jnp.float32)
        r_s = 1.0 / (s_tot + EPS)
        acc_h = jnp.zeros((8, 1), jnp.float32)
        for k in range(26):
            lo_s = jnp.sum(jnp.bitwise_and(longs[k], m16),
                           axis=1, keepdims=True)
            hi_s = jnp.sum(jnp.bitwise_and(
                jax.lax.shift_right_logical(longs[k], 8), m16),
                axis=1, keepdims=True)
            for half in (jnp.bitwise_and(lo_s, 0xFFFF),
                         jnp.bitwise_and(hi_s, 0xFFFF),
                         jax.lax.shift_right_logical(lo_s, 16),
                         jax.lax.shift_right_logical(hi_s, 16)):
                p = half.astype(jnp.float32) * r_s
                acc_h = acc_h + p * jnp.log(p + EPS)
        ent_ref[pl.ds(g8, 8), :] = -acc_h
        return carry

    jax.lax.fori_loop(0, r // 8, grp_body, 0)
    passed = ent_ref[...] >= ENT_THRESH

    comp = jnp.dot(x, st_ref[...], preferred_element_type=jnp.float32)
    filt = jnp.where(passed, comp, 0.0)
    state = jnp.dot(filt, adj_ref[...], preferred_element_type=jnp.float32)

    pre1 = jnp.dot(state, w1_ref[...], preferred_element_type=jnp.float32) + b1_ref[...]
    relu_mask = (pre1 > 0).astype(jnp.float32)
    c_row = (jnp.sum(w2_ref[...], axis=1) * scale_ref[0]).reshape(1, -1)
    g = jnp.dot(relu_mask * c_row, w1t_ref[...], preferred_element_type=jnp.float32)
    s_adv = state - 0.1 * jnp.sign(g)
    hid = jnp.maximum(
        jnp.dot(s_adv, w1_ref[...], preferred_element_type=jnp.float32) + b1_ref[...],
        0.0)
    logits = jnp.dot(hid, w2_ref[...], preferred_element_type=jnp.float32) + b2_ref[...]
    mx = jnp.max(logits, axis=1, keepdims=True)
    ex = jnp.exp(logits - mx)
    out_ref[...] = ex / jnp.sum(ex, axis=1, keepdims=True)


def _expm_body(adj_ref, out_ref):
    a = adj_ref[...]
    n = a.shape[0]
    t = (a * a) * (2.0 ** -SQUARINGS)
    eye = (jax.lax.broadcasted_iota(jnp.int32, (n, n), 0) ==
           jax.lax.broadcasted_iota(jnp.int32, (n, n), 1)).astype(jnp.float32)
    p = eye + t * (1.0 / 6.0)
    for k in (5.0, 4.0, 3.0, 2.0):
        p = eye + jnp.dot(t, p, preferred_element_type=jnp.float32,
                          precision=jax.lax.Precision.HIGHEST) * (1.0 / k)
    e = jnp.dot(t, p, preferred_element_type=jnp.float32,
                precision=jax.lax.Precision.HIGHEST)
    for _ in range(SQUARINGS):
        e = 2.0 * e + jnp.dot(e, e, preferred_element_type=jnp.float32,
                              precision=jax.lax.Precision.HIGHEST)
    tr_col = jnp.sum(e * eye, axis=1, keepdims=True)
    out_ref[...] = jnp.sum(tr_col, axis=0, keepdims=True)


def kernel(event_stream, sensing_matrix, adjacency, w1, b1, w2, b2):
    b_dim, d_dim = event_stream.shape
    n_dim = sensing_matrix.shape[0]
    h_dim = w1.shape[1]
    a_dim = w2.shape[1]
    r = 256 if b_dim % 256 == 0 else b_dim
    grid = b_dim // r

    st = sensing_matrix.T
    w1t = w1.T
    b1r = b1.reshape(1, h_dim)
    b2r = b2.reshape(1, a_dim)
    scale = jnp.full((1,), 1.0 / (b_dim * a_dim), jnp.float32)

    probs = pl.pallas_call(
        _main_body,
        grid=(grid,),
        in_specs=[
            pl.BlockSpec((r, d_dim), lambda i: (i, 0)),
            pl.BlockSpec((d_dim, n_dim), lambda i: (0, 0)),
            pl.BlockSpec((n_dim, n_dim), lambda i: (0, 0)),
            pl.BlockSpec((n_dim, h_dim), lambda i: (0, 0)),
            pl.BlockSpec((h_dim, n_dim), lambda i: (0, 0)),
            pl.BlockSpec((1, h_dim), lambda i: (0, 0)),
            pl.BlockSpec((h_dim, a_dim), lambda i: (0, 0)),
            pl.BlockSpec((1, a_dim), lambda i: (0, 0)),
            pl.BlockSpec(memory_space=pltpu.SMEM),
        ],
        out_specs=pl.BlockSpec((r, a_dim), lambda i: (i, 0)),
        out_shape=jax.ShapeDtypeStruct((b_dim, a_dim), jnp.float32),
        scratch_shapes=[pltpu.VMEM((r, 1), jnp.float32)],
        compiler_params=pltpu.CompilerParams(
            dimension_semantics=("arbitrary",),
            vmem_limit_bytes=56 * 1024 * 1024,
        ),
        name="acie_fused_main",
    )(event_stream, st, adjacency, w1, w1t, b1r, w2, b2r, scale)

    tr = pl.pallas_call(
        _expm_body,
        out_shape=jax.ShapeDtypeStruct((1, 1), jnp.float32),
        name="acie_expm_trace",
    )(adjacency)

    return (probs, tr[0, 0])


# speedup vs baseline: 1.1002x; 1.0275x over previous
import jax
import jax.numpy as jnp
from jax.experimental import pallas as pl
from jax.experimental.pallas import tpu as pltpu

BINS = 100
LO, HI = -5.0, 5.0
WIDTH = (HI - LO) / BINS
ENT_THRESH = 2.5
EPS = 1e-9
SQUARINGS = 5


def _main_body(e_ref, st_ref, adj_ref, w1_ref, w1t_ref, b1_ref, w2_ref,
               b2_ref, scale_ref, out_ref, ent_ref):
    x = e_ref[...]
    r, d = x.shape
    n_chunks = d // 128

    nib = jnp.int32(0x0F0F0F0F)
    m16 = jnp.int32(0x00FF00FF)

    def grp_body(gi, carry):
        g8 = pl.multiple_of(gi * 8, 8)
        longs = [jnp.zeros((8, 128), jnp.int32) for _ in range(26)]
        vcnt = jnp.zeros((8, 128), jnp.int32)
        for r0 in range(0, n_chunks, 15):
            saccs = [jnp.zeros((8, 128), jnp.int32) for _ in range(13)]
            for c in range(r0, min(r0 + 15, n_chunks)):
                xc = e_ref[pl.ds(g8, 8), c * 128:(c + 1) * 128]
                t = (xc - LO) * jnp.float32(1.0 / WIDTH)
                ii = t.astype(jnp.int32)
                valid = jnp.abs(xc) <= HI
                idxs = jnp.where(valid, jnp.minimum(ii, BINS - 1), 1000)
                qq = jax.lax.shift_right_logical(idxs, 3)
                sv = jax.lax.shift_left(
                    jnp.int32(1),
                    jax.lax.shift_left(jnp.bitwise_and(idxs, 7), 2))
                vcnt = vcnt + jnp.where(valid, 1, 0)
                for k in range(13):
                    saccs[k] = saccs[k] + jnp.where(qq == k, sv, 0)
            for k in range(13):
                longs[2 * k] = longs[2 * k] + jnp.bitwise_and(saccs[k], nib)
                longs[2 * k + 1] = longs[2 * k + 1] + jnp.bitwise_and(
                    jax.lax.shift_right_logical(saccs[k], 4), nib)
        s_tot = jnp.sum(vcnt, axis=1, keepdims=True).astype(jnp.float32)
        r_s = 1.0 / (s_tot + EPS)
        acc_h = jnp.zeros((8, 1), jnp.float32)
        for k in range(26):
            lo_s = jnp.sum(jnp.bitwise_and(longs[k], m16),
                           axis=1, keepdims=True)
            hi_s = jnp.sum(jnp.bitwise_and(
                jax.lax.shift_right_logical(longs[k], 8), m16),
                axis=1, keepdims=True)
            for half in (jnp.bitwise_and(lo_s, 0xFFFF),
                         jnp.bitwise_and(hi_s, 0xFFFF),
                         jax.lax.shift_right_logical(lo_s, 16),
                         jax.lax.shift_right_logical(hi_s, 16)):
                p = half.astype(jnp.float32) * r_s
                acc_h = acc_h + p * jnp.log(p + EPS)
        ent_ref[pl.ds(g8, 8), :] = -acc_h
        return carry

    jax.lax.fori_loop(0, r // 8, grp_body, 0)
    passed = ent_ref[...] >= ENT_THRESH

    comp = jnp.dot(x, st_ref[...], preferred_element_type=jnp.float32)
    filt = jnp.where(passed, comp, 0.0)
    state = jnp.dot(filt, adj_ref[...], preferred_element_type=jnp.float32)

    pre1 = jnp.dot(state, w1_ref[...], preferred_element_type=jnp.float32) + b1_ref[...]
    relu_mask = (pre1 > 0).astype(jnp.float32)
    c_row = (jnp.sum(w2_ref[...], axis=1) * scale_ref[0]).reshape(1, -1)
    g = jnp.dot(relu_mask * c_row, w1t_ref[...], preferred_element_type=jnp.float32)
    s_adv = state - 0.1 * jnp.sign(g)
    hid = jnp.maximum(
        jnp.dot(s_adv, w1_ref[...], preferred_element_type=jnp.float32) + b1_ref[...],
        0.0)
    logits = jnp.dot(hid, w2_ref[...], preferred_element_type=jnp.float32) + b2_ref[...]
    mx = jnp.max(logits, axis=1, keepdims=True)
    ex = jnp.exp(logits - mx)
    out_ref[...] = ex / jnp.sum(ex, axis=1, keepdims=True)


def _expm_body(adj_ref, out_ref):
    a = adj_ref[...]
    n = a.shape[0]
    t = (a * a) * (2.0 ** -SQUARINGS)
    eye = (jax.lax.broadcasted_iota(jnp.int32, (n, n), 0) ==
           jax.lax.broadcasted_iota(jnp.int32, (n, n), 1)).astype(jnp.float32)
    p = eye + t * (1.0 / 6.0)
    for k in (5.0, 4.0, 3.0, 2.0):
        p = eye + jnp.dot(t, p, preferred_element_type=jnp.float32,
                          precision=jax.lax.Precision.HIGHEST) * (1.0 / k)
    e = jnp.dot(t, p, preferred_element_type=jnp.float32,
                precision=jax.lax.Precision.HIGHEST)
    for _ in range(SQUARINGS):
        e = 2.0 * e + jnp.dot(e, e, preferred_element_type=jnp.float32,
                              precision=jax.lax.Precision.HIGHEST)
    tr_col = jnp.sum(e * eye, axis=1, keepdims=True)
    out_ref[...] = jnp.sum(tr_col, axis=0, keepdims=True)


def kernel(event_stream, sensing_matrix, adjacency, w1, b1, w2, b2):
    b_dim, d_dim = event_stream.shape
    n_dim = sensing_matrix.shape[0]
    h_dim = w1.shape[1]
    a_dim = w2.shape[1]
    r = 512 if b_dim % 512 == 0 else b_dim
    grid = b_dim // r

    st = sensing_matrix.T
    w1t = w1.T
    b1r = b1.reshape(1, h_dim)
    b2r = b2.reshape(1, a_dim)
    scale = jnp.full((1,), 1.0 / (b_dim * a_dim), jnp.float32)

    probs = pl.pallas_call(
        _main_body,
        grid=(grid,),
        in_specs=[
            pl.BlockSpec((r, d_dim), lambda i: (i, 0)),
            pl.BlockSpec((d_dim, n_dim), lambda i: (0, 0)),
            pl.BlockSpec((n_dim, n_dim), lambda i: (0, 0)),
            pl.BlockSpec((n_dim, h_dim), lambda i: (0, 0)),
            pl.BlockSpec((h_dim, n_dim), lambda i: (0, 0)),
            pl.BlockSpec((1, h_dim), lambda i: (0, 0)),
            pl.BlockSpec((h_dim, a_dim), lambda i: (0, 0)),
            pl.BlockSpec((1, a_dim), lambda i: (0, 0)),
            pl.BlockSpec(memory_space=pltpu.SMEM),
        ],
        out_specs=pl.BlockSpec((r, a_dim), lambda i: (i, 0)),
        out_shape=jax.ShapeDtypeStruct((b_dim, a_dim), jnp.float32),
        scratch_shapes=[pltpu.VMEM((r, 1), jnp.float32)],
        compiler_params=pltpu.CompilerParams(
            dimension_semantics=("arbitrary",),
            vmem_limit_bytes=56 * 1024 * 1024,
        ),
        name="acie_fused_main",
    )(event_stream, st, adjacency, w1, w1t, b1r, w2, b2r, scale)

    tr = pl.pallas_call(
        _expm_body,
        out_shape=jax.ShapeDtypeStruct((1, 1), jnp.float32),
        name="acie_expm_trace",
    )(adjacency)

    return (probs, tr[0, 0])
